# Initial kernel scaffold; baseline (speedup 1.0000x reference)
import jax
import jax.numpy as jnp
from jax.experimental import pallas as pl


def kernel(x_cat, x_cont, tables, W, b):
    raise NotImplementedError("write your pallas kernel here")


# speedup vs baseline: 8.7380x; 8.7380x over previous
import jax
import jax.numpy as jnp
from jax import lax
from jax.experimental import pallas as pl
from jax.experimental.pallas import tpu as pltpu
from jax.experimental.pallas import tpu_sc as plsc

_F = 26
_V = 100000
_E = 32
_L = 16
_NC = 2
_NS = 16
_NW = _NC * _NS
_B = 16384
_RPW = _B // _NW
_R = 32
_NCH = _RPW // _R
_CIDX = _R * _F
_SG = 104
_NSG = _CIDX // _SG
_IDXW = _RPW * _F


def _body(tables_ref, xcat_ref, xcont_ref, w_ref, wc_ref, out_ref,
          idx_v, rows0, rows1, xcont_v, out_v, w_v, wc_v, sem0, sem1):
    wid = lax.axis_index("s") * _NC + lax.axis_index("c")
    base = wid * _RPW
    ibase = base * _F

    pltpu.sync_copy(xcat_ref.at[pl.ds(ibase, _IDXW)], idx_v)
    pltpu.sync_copy(xcont_ref.at[pl.ds(base, _RPW)], xcont_v)
    pltpu.sync_copy(w_ref, w_v)
    pltpu.sync_copy(wc_ref, wc_v)

    lanes = lax.iota(jnp.int32, _L)

    def build_idx(j, carry):
        sl = pl.ds(j * _L, _L)
        f = (j * _L + lanes) % _F
        idx_v[sl] = idx_v[sl] + f * _V
        return carry

    lax.fori_loop(0, _IDXW // _L, build_idx, 0)

    rows = (rows0, rows1)
    sems = (sem0, sem1)

    def fire(ci, k):
        for g in range(_NSG):
            isl = pl.ds(ci * _CIDX + g * _SG, _SG)
            pltpu.async_copy(tables_ref.at[idx_v.at[isl]],
                             rows[k].at[pl.ds(g * _SG, _SG)], sems[k])

    def drain(ci, k):
        for g in range(_NSG):
            isl = pl.ds(ci * _CIDX + g * _SG, _SG)
            pltpu.make_async_copy(tables_ref.at[idx_v.at[isl]],
                                  rows[k].at[pl.ds(g * _SG, _SG)], sems[k]).wait()

    def lane_sum(v):
        for s in (8, 4, 2, 1):
            idx = (lanes ^ s)[:, None]
            dn = lax.GatherDimensionNumbers(
                offset_dims=(), collapsed_slice_dims=(0,), start_index_map=(0,))
            v = v + lax.gather(v, idx, dn, (1,),
                               mode=lax.GatherScatterMode.PROMISE_IN_BOUNDS)
        return v

    wc = wc_v[...]
    w0 = [w_v[pl.ds(f * _E, _L)] for f in range(_F)]
    w1 = [w_v[pl.ds(f * _E + _L, _L)] for f in range(_F)]

    def compute(ci, k):
        rbuf = rows[k]

        def group_body(g, carry):
            def row_body(r2, totvec):
                r = g * _L + r2
                acc0 = xcont_v[ci * _R + r, :] * wc
                acc1 = jnp.zeros((_L,), jnp.float32)
                rb = r * _F
                for f in range(_F):
                    acc0 = acc0 + rbuf[rb + f, pl.ds(0, _L)] * w0[f]
                    acc1 = acc1 + rbuf[rb + f, pl.ds(_L, _L)] * w1[f]
                tot = lane_sum(acc0 + acc1)
                return jnp.where(lanes == r2, tot, totvec)

            totvec = lax.fori_loop(0, _L, row_body,
                                   jnp.zeros((_L,), jnp.float32))
            out_v[pl.ds(ci * _R + g * _L, _L)] = 1.0 / (1.0 + jnp.exp(-totvec))
            return carry

        lax.fori_loop(0, _R // _L, group_body, 0)

    fire(0, 0)

    def outer(c2, carry):
        for k in range(2):
            i = c2 * 2 + k
            drain(i, k)

            @pl.when(i + 1 < _NCH)
            def _():
                fire(i + 1, k ^ 1)

            compute(i, k)
        return carry

    lax.fori_loop(0, _NCH // 2, outer, 0)

    pltpu.sync_copy(out_v, out_ref.at[pl.ds(base, _RPW)])


@jax.jit
def _run(tables_flat, xcat_flat, xcont_pad, w_main, wc_pad):
    k = pl.kernel(
        _body,
        out_type=jax.ShapeDtypeStruct((_B,), jnp.float32),
        mesh=plsc.VectorSubcoreMesh(core_axis_name="c", subcore_axis_name="s",
                                    num_cores=_NC, num_subcores=_NS),
        compiler_params=pltpu.CompilerParams(use_tc_tiling_on_sc=False),
        scratch_types=[
            pltpu.VMEM((_IDXW,), jnp.int32),
            pltpu.VMEM((_CIDX, _E), jnp.float32),
            pltpu.VMEM((_CIDX, _E), jnp.float32),
            pltpu.VMEM((_RPW, _L), jnp.float32),
            pltpu.VMEM((_RPW,), jnp.float32),
            pltpu.VMEM((_F * _E,), jnp.float32),
            pltpu.VMEM((_L,), jnp.float32),
            pltpu.SemaphoreType.DMA,
            pltpu.SemaphoreType.DMA,
        ],
    )
    return k(tables_flat, xcat_flat, xcont_pad, w_main, wc_pad)


def kernel(x_cat, x_cont, tables, W, b):
    bsz = x_cat.shape[0]
    tables_flat = tables.reshape(_F * _V, _E)
    xcat_flat = x_cat.reshape(-1)
    xcont_pad = jnp.concatenate(
        [x_cont, jnp.ones((bsz, 1), jnp.float32), jnp.zeros((bsz, 2), jnp.float32)],
        axis=1)
    w_main = W[: _F * _E, 0]
    wc_pad = jnp.concatenate([W[_F * _E:, 0], b, jnp.zeros((2,), jnp.float32)])
    out = _run(tables_flat, xcat_flat, xcont_pad, w_main, wc_pad)
    return out.reshape(bsz, 1)
